# baseline (speedup 1.0000x reference)
import functools

import jax
import jax.numpy as jnp
from jax import lax
from jax.experimental import pallas as pl
from jax.experimental.pallas import tpu as pltpu
from jax.experimental.pallas import tpu_sc as plsc

N = 131072
C = 20
K = 1024
R = 48
NTILES = 32
PT = N // NTILES
SUB = 1024
NSUB = PT // SUB
BB = 4096


def _tc_bin_body(preds_ref, labels_ref, binsT_ref, fgrow_ref, fgb_ref):
    x = preds_ref[...]
    m = jnp.max(x, axis=1, keepdims=True)
    t = jnp.exp(x - m)
    s = jnp.sum(t, axis=1, keepdims=True)
    f = jnp.abs(jnp.float32(K) - t * (jnp.float32(K) / s))
    b = jnp.minimum(f.astype(jnp.int32), K - 1)
    binsT_ref[...] = b.T
    lab = labels_ref[...]
    onehot = (lab[:, None] == lax.broadcasted_iota(jnp.int32, (1, C), 1))
    fgb_ref[...] = jnp.sum(jnp.where(onehot, b, 0), axis=1)
    fgrow_ref[...] = lab + C


def _tc_bin(preds, labels):
    nblk = N // BB
    return pl.pallas_call(
        _tc_bin_body,
        grid=(nblk,),
        in_specs=[
            pl.BlockSpec((BB, C), lambda i: (i, 0)),
            pl.BlockSpec((BB,), lambda i: (i,)),
        ],
        out_specs=[
            pl.BlockSpec((C, BB), lambda i: (0, i)),
            pl.BlockSpec((BB,), lambda i: (i,)),
            pl.BlockSpec((BB,), lambda i: (i,)),
        ],
        out_shape=[
            jax.ShapeDtypeStruct((C, N), jnp.int32),
            jax.ShapeDtypeStruct((N,), jnp.int32),
            jax.ShapeDtypeStruct((N,), jnp.int32),
        ],
    )(preds, labels)


def _sc_body(binsT_hbm, fgrow_hbm, fgb_hbm, out_hbm, bbuf, rbuf, cbuf, hist,
             rowidx, shared, sem1, sem2, sem3):
    core = lax.axis_index("c")
    sid = lax.axis_index("s")

    zeros16 = jnp.zeros((16,), jnp.float32)
    ones16 = jnp.ones((16,), jnp.float32)
    lane = lax.iota(jnp.int32, 16)

    def _zrow(i, carry):
        for j in range(K // 16):
            hist[i, pl.ds(j * 16, 16)] = zeros16
        return carry
    lax.fori_loop(0, R, _zrow, 0)
    for j in range(R // 16):
        rowidx[pl.ds(j * 16, 16)] = lane + (j * 16)

    @pl.when(sid == 0)
    def _():
        pltpu.sync_copy(hist, shared)

    def _one_group(p0, buf):
        for c in range(C):
            b = bbuf[buf, c, pl.ds(p0, 16)]
            plsc.addupdate_scatter(
                hist, [jnp.full((16,), c, jnp.int32), b], ones16)
        fr = rbuf[buf, pl.ds(p0, 16)]
        fc = cbuf[buf, pl.ds(p0, 16)]
        plsc.addupdate_scatter(hist, [fr, fc], ones16)

    wid = core * 16 + sid

    def _start(s):
        b = s % 2
        base = wid * PT + s * SUB
        h1 = pltpu.async_copy(
            binsT_hbm.at[:, pl.ds(base, SUB)], bbuf.at[b], sem1)
        h2 = pltpu.async_copy(fgrow_hbm.at[pl.ds(base, SUB)], rbuf.at[b], sem2)
        h3 = pltpu.async_copy(fgb_hbm.at[pl.ds(base, SUB)], cbuf.at[b], sem3)
        return h1, h2, h3

    h = _start(0)
    for sidx in range(NSUB):
        hn = _start(sidx + 1) if sidx + 1 < NSUB else None
        for hh in h:
            hh.wait()
        buf = sidx % 2

        @plsc.parallel_loop(0, SUB // 16, 1, unroll=2)
        def _body(g):
            _one_group(g * 16, buf)
        h = hn

    plsc.subcore_barrier()
    pltpu.sync_copy(hist, shared.at[rowidx], add=True)
    plsc.subcore_barrier()
    @pl.when(sid == 0)
    def _():
        pltpu.sync_copy(shared, out_hbm.at[core])


@functools.partial(
    pl.kernel,
    out_type=jax.ShapeDtypeStruct((2, R, K), jnp.float32),
    mesh=plsc.VectorSubcoreMesh(core_axis_name="c", subcore_axis_name="s"),
    compiler_params=pltpu.CompilerParams(
        needs_layout_passes=False, use_tc_tiling_on_sc=False),
    scratch_types=[
        pltpu.VMEM((2, C, SUB), jnp.int32),
        pltpu.VMEM((2, SUB), jnp.int32),
        pltpu.VMEM((2, SUB), jnp.int32),
        pltpu.VMEM((R, K), jnp.float32),
        pltpu.VMEM((R,), jnp.int32),
        pltpu.VMEM_SHARED((R, K), jnp.float32),
        pltpu.SemaphoreType.DMA,
        pltpu.SemaphoreType.DMA,
        pltpu.SemaphoreType.DMA,
    ],
)
def _sc_hist(binsT_hbm, fgrow_hbm, fgb_hbm, out_hbm, bbuf, rbuf, cbuf, hist,
             rowidx, shared, sem1, sem2, sem3):
    _sc_body(binsT_hbm, fgrow_hbm, fgb_hbm, out_hbm, bbuf, rbuf, cbuf, hist,
             rowidx, shared, sem1, sem2, sem3)


def _tc_body(hist_ref, out_ref):
    tot = jnp.sum(hist_ref[...], axis=0)
    cnt = tot[:C, :]
    fg = tot[C:2 * C, :]
    ir = lax.broadcasted_iota(jnp.int32, (K, K), 0)
    ic = lax.broadcasted_iota(jnp.int32, (K, K), 1)
    M = (ir >= ic).astype(jnp.float32)
    dn = (((1,), (0,)), ((), ()))
    Nk = lax.dot_general(cnt, M, dn, preferred_element_type=jnp.float32)
    Fk = lax.dot_general(fg, M, dn, preferred_element_type=jnp.float32)
    G = Fk[:, 0:1]
    denom = G + Nk - Fk
    I = jnp.where(denom > 0, 1.0 - (G - Fk) / denom, 0.0)
    loss_c = (jnp.sum(I, axis=1, keepdims=True) - 0.5 * I[:, 0:1]) * (1.0 / K)
    present = (G > 0).astype(jnp.float32)
    loss = jnp.sum(loss_c * present) / jnp.maximum(jnp.sum(present), 1.0)
    out_ref[...] = jnp.broadcast_to(loss, (1, 1))


def _tc_finish(hist):
    return pl.pallas_call(
        _tc_body,
        out_shape=jax.ShapeDtypeStruct((1, 1), jnp.float32),
    )(hist)


def kernel(preds, labels):
    labels = labels.astype(jnp.int32)
    binsT, fgrow, fgb = _tc_bin(preds, labels)
    hist = _sc_hist(binsT, fgrow, fgb)
    return _tc_finish(hist)[0, 0]


# speedup vs baseline: 3.6762x; 3.6762x over previous
import functools

import jax
import jax.numpy as jnp
from jax import lax
from jax.experimental import pallas as pl
from jax.experimental.pallas import tpu as pltpu
from jax.experimental.pallas import tpu_sc as plsc

N = 131072
C = 20
K = 1024
R = 48
NTILES = 32
PT = N // NTILES
SUB = 1024
NSUB = PT // SUB


def _tree(xs, op):
    xs = list(xs)
    while len(xs) > 1:
        nxt = [op(xs[i], xs[i + 1]) for i in range(0, len(xs) - 1, 2)]
        if len(xs) % 2:
            nxt.append(xs[-1])
        xs = nxt
    return xs[0]


def _sc_body(predsT_hbm, labels_hbm, out_hbm, pbuf, lbuf, hist, rowidx,
             shared, semp, seml):
    core = lax.axis_index("c")
    sid = lax.axis_index("s")

    zeros16 = jnp.zeros((16,), jnp.float32)
    ones16 = jnp.ones((16,), jnp.float32)
    lane = lax.iota(jnp.int32, 16)

    def _zrow(i, carry):
        for j in range(K // 16):
            hist[i, pl.ds(j * 16, 16)] = zeros16
        return carry
    lax.fori_loop(0, R, _zrow, 0)
    for j in range(R // 16):
        rowidx[pl.ds(j * 16, 16)] = lane + (j * 16)

    @pl.when(sid == 0)
    def _():
        pltpu.sync_copy(hist, shared)

    def _one_group(p0, buf):
        lab = lbuf[buf, pl.ds(p0, 16)]
        v = [pbuf[buf, c, pl.ds(p0, 16)] for c in range(C)]
        m = _tree(v, jnp.maximum)
        t = [jnp.exp(v[c] - m) for c in range(C)]
        s = _tree(t, lambda a, b: a + b)
        kr = jnp.float32(K) / s
        fgbin = jnp.zeros((16,), jnp.int32)
        for c in range(C):
            f = jnp.abs(jnp.float32(K) - t[c] * kr)
            b = jnp.minimum(f.astype(jnp.int32), K - 1)
            plsc.addupdate_scatter(
                hist, [jnp.full((16,), c, jnp.int32), b], ones16)
            fgbin = jnp.where(lab == c, b, fgbin)
        plsc.addupdate_scatter(hist, [lab + C, fgbin], ones16)

    wid = core * 16 + sid

    def _start(s):
        b = s % 2
        base = wid * PT + s * SUB
        h1 = pltpu.async_copy(
            predsT_hbm.at[:, pl.ds(base, SUB)], pbuf.at[b], semp)
        h2 = pltpu.async_copy(
            labels_hbm.at[pl.ds(base, SUB)], lbuf.at[b], seml)
        return h1, h2

    h = _start(0)
    for sidx in range(NSUB):
        hn = _start(sidx + 1) if sidx + 1 < NSUB else None
        h[0].wait()
        h[1].wait()
        buf = sidx % 2

        @plsc.parallel_loop(0, SUB // 16, 1, unroll=4)
        def _body(g):
            _one_group(g * 16, buf)
        h = hn

    plsc.subcore_barrier()
    pltpu.sync_copy(hist, shared.at[rowidx], add=True)
    plsc.subcore_barrier()
    @pl.when(sid == 0)
    def _():
        pltpu.sync_copy(shared, out_hbm.at[core])


@functools.partial(
    pl.kernel,
    out_type=jax.ShapeDtypeStruct((2, R, K), jnp.float32),
    mesh=plsc.VectorSubcoreMesh(core_axis_name="c", subcore_axis_name="s"),
    compiler_params=pltpu.CompilerParams(
        needs_layout_passes=False, use_tc_tiling_on_sc=False),
    scratch_types=[
        pltpu.VMEM((2, C, SUB), jnp.float32),
        pltpu.VMEM((2, SUB), jnp.int32),
        pltpu.VMEM((R, K), jnp.float32),
        pltpu.VMEM((R,), jnp.int32),
        pltpu.VMEM_SHARED((R, K), jnp.float32),
        pltpu.SemaphoreType.DMA,
        pltpu.SemaphoreType.DMA,
    ],
)
def _sc_hist(predsT_hbm, labels_hbm, out_hbm, pbuf, lbuf, hist, rowidx,
             shared, semp, seml):
    _sc_body(predsT_hbm, labels_hbm, out_hbm, pbuf, lbuf, hist, rowidx,
             shared, semp, seml)


def _tc_body(hist_ref, out_ref):
    tot = jnp.sum(hist_ref[...], axis=0)
    cnt = tot[:C, :]
    fg = tot[C:2 * C, :]
    ir = lax.broadcasted_iota(jnp.int32, (K, K), 0)
    ic = lax.broadcasted_iota(jnp.int32, (K, K), 1)
    M = (ir >= ic).astype(jnp.float32)
    dn = (((1,), (0,)), ((), ()))
    Nk = lax.dot_general(cnt, M, dn, preferred_element_type=jnp.float32)
    Fk = lax.dot_general(fg, M, dn, preferred_element_type=jnp.float32)
    G = Fk[:, 0:1]
    denom = G + Nk - Fk
    I = jnp.where(denom > 0, 1.0 - (G - Fk) / denom, 0.0)
    loss_c = (jnp.sum(I, axis=1, keepdims=True) - 0.5 * I[:, 0:1]) * (1.0 / K)
    present = (G > 0).astype(jnp.float32)
    loss = jnp.sum(loss_c * present) / jnp.maximum(jnp.sum(present), 1.0)
    out_ref[...] = jnp.broadcast_to(loss, (1, 1))


def _tc_finish(hist):
    return pl.pallas_call(
        _tc_body,
        out_shape=jax.ShapeDtypeStruct((1, 1), jnp.float32),
    )(hist)


def kernel(preds, labels):
    labels = labels.astype(jnp.int32)
    hist = _sc_hist(jnp.swapaxes(preds, 0, 1), labels)
    return _tc_finish(hist)[0, 0]
